# baseline (speedup 1.0000x reference)
import functools

import jax
import jax.numpy as jnp
from jax import lax
from jax.experimental import pallas as pl
from jax.experimental.pallas import tpu as pltpu
from jax.experimental.pallas import tpu_sc as plsc

GRID = 128
CELLS = GRID * GRID
C = 196
CP = 208
HALF = CP // 2
NPLANES = 3
NSUB = 16
PCHUNK = 64


def _pre_body(consts_ref, gr_ref, p_ref, w_ref, idx_ref):
    Bn = gr_ref.shape[2]
    npts = pl.num_programs(1) * Bn
    rmod = npts % C
    n0 = pl.program_id(1) * Bn
    a = gr_ref[0]
    ci = lax.broadcasted_iota(jnp.int32, (C, Bn), 0)
    ni = n0 + lax.broadcasted_iota(jnp.int32, (C, Bn), 1)
    q = lax.rem(ni + rmod * ci, C)
    s0 = consts_ref[0]
    o0 = consts_ref[1]
    s1 = consts_ref[2]
    o1 = consts_ref[3]
    s2 = consts_ref[4]
    o2 = consts_ref[5]
    bv = jnp.where(q == 0, a * s0 + o0,
         jnp.where(q == 1, a * s1 + o1,
         jnp.where(q == 2, a * s2 + o2, a)))
    p4 = p_ref[0]
    alpha = jax.nn.sigmoid(p4[3:4, :])
    w_ref[0, 0:C, :] = bv * alpha
    w_ref[0, C:C + 1, :] = alpha
    w_ref[0, C + 1:CP, :] = jnp.zeros((CP - C - 1, Bn), jnp.float32)

    x = p4[0:1, :] * s0 + o0
    y = p4[1:2, :] * s1 + o1
    z = p4[2:3, :] * s2 + o2

    def cell(u):
        g = ((u * 0.5 + 0.5) * (GRID - 1)).astype(jnp.int32)
        return jnp.clip(g, 0, GRID - 1)

    gx, gy, gz = cell(x), cell(y), cell(z)
    idx_ref[0, 0:1, :] = gx * GRID + gy
    idx_ref[0, 1:2, :] = gx * GRID + gz
    idx_ref[0, 2:3, :] = gy * GRID + gz


def _tr_body(w_ref, t_ref):
    a = w_ref[0, 0]
    apad = jnp.concatenate(
        [a, jnp.zeros((128 - HALF, a.shape[1]), jnp.float32)], axis=0)
    t = jnp.transpose(apad)
    t_ref[0, 0] = t[:, 0:HALF]


def _post_body(a0_ref, a1_ref, o0_ref, o1_ref):
    m0 = a0_ref[0]
    m1 = a1_ref[0]
    w = jnp.maximum(m1[:, C - HALF:C - HALF + 1], 1e-6)
    o0_ref[0] = m0 / w
    o1_ref[0] = m1[:, 0:C - HALF] / w


def _make_sc_scatter(nbatch, npts):
    pts_per_sub = npts // NSUB
    nchunks = pts_per_sub // PCHUNK
    rows_per_sub = CELLS // NSUB
    nrounds = nbatch * NPLANES
    mesh = plsc.VectorSubcoreMesh(core_axis_name="c", subcore_axis_name="s")

    npairs = nchunks // 2

    @functools.partial(
        pl.kernel,
        mesh=mesh,
        compiler_params=pltpu.CompilerParams(use_tc_tiling_on_sc=False),
        out_type=jax.ShapeDtypeStruct((2, nbatch, NPLANES, CELLS, HALF),
                                      jnp.float32),
        scratch_types=[
            pltpu.VMEM_SHARED((CELLS, HALF), jnp.float32),
            pltpu.VMEM((PCHUNK,), jnp.int32),
            pltpu.VMEM((PCHUNK, HALF), jnp.float32),
            pltpu.VMEM((PCHUNK,), jnp.int32),
            pltpu.VMEM((PCHUNK, HALF), jnp.float32),
            pltpu.SemaphoreType.DMA,
            pltpu.SemaphoreType.DMA,
        ],
    )
    def sc_scatter(wt_hbm, idx_hbm, z_hbm, out_hbm, acc,
                   idxa, rowsa, idxb, rowsb, sema, semb):
        cid = lax.axis_index("c")
        sid = lax.axis_index("s")
        r0 = sid * rows_per_sub
        pbase = sid * pts_per_sub

        def load(k, idxv, rows, sem, b, p):
            base = pbase + k * PCHUNK
            pltpu.async_copy(idx_hbm.at[b, p, 0, pl.ds(base, PCHUNK)],
                             idxv, sem)
            pltpu.async_copy(wt_hbm.at[cid, b, pl.ds(base, PCHUNK), :],
                             rows, sem)

        def drain(idxv, rows, sem, b, p):
            pltpu.make_async_copy(idx_hbm.at[b, p, 0, pl.ds(pbase, PCHUNK)],
                                  idxv, sem).wait()
            pltpu.make_async_copy(wt_hbm.at[cid, b, pl.ds(pbase, PCHUNK), :],
                                  rows, sem).wait()

        def round_body(r, carry):
            b = r // NPLANES
            p = lax.rem(r, NPLANES)
            pltpu.sync_copy(z_hbm.at[pl.ds(r0, rows_per_sub), :],
                            acc.at[pl.ds(r0, rows_per_sub), :])
            plsc.subcore_barrier()
            load(0, idxa, rowsa, sema, b, p)

            def pair(i, carry2):
                load(2 * i + 1, idxb, rowsb, semb, b, p)
                drain(idxa, rowsa, sema, b, p)
                pltpu.sync_copy(rowsa, acc.at[idxa], add=True)

                @pl.when(i + 1 < npairs)
                def _():
                    load(2 * i + 2, idxa, rowsa, sema, b, p)

                drain(idxb, rowsb, semb, b, p)
                pltpu.sync_copy(rowsb, acc.at[idxb], add=True)
                return carry2

            lax.fori_loop(0, npairs, pair, 0)
            plsc.subcore_barrier()
            pltpu.sync_copy(
                acc.at[pl.ds(r0, rows_per_sub), :],
                out_hbm.at[cid, b, p, pl.ds(r0, rows_per_sub), :])
            return carry

        lax.fori_loop(0, nrounds, round_body, 0)

    return sc_scatter


def kernel(GS_feats, scene_bounds):
    nbatch, npts, nchan = GS_feats.shape
    sb = scene_bounds.astype(jnp.float32)
    s0 = 2.0 / (sb[1] - sb[0])
    o0 = -2.0 * sb[0] / (sb[1] - sb[0]) - 1.0
    s1 = 2.0 / (sb[3] - sb[2])
    o1 = -2.0 * sb[2] / (sb[3] - sb[2]) - 1.0
    s2 = 2.0 / (sb[5] - sb[4])
    o2 = -2.0 * sb[4] / (sb[5] - sb[4]) - 1.0
    consts = jnp.stack([s0, o0, s1, o1, s2, o2,
                        jnp.float32(0.0), jnp.float32(0.0)])

    gr = GS_feats.reshape(nbatch, nchan, npts)
    p4 = jnp.transpose(GS_feats[:, :, 0:4], (0, 2, 1))

    Bn = 512
    wpad, idx = pl.pallas_call(
        _pre_body,
        grid=(nbatch, npts // Bn),
        in_specs=[
            pl.BlockSpec(memory_space=pltpu.SMEM),
            pl.BlockSpec((1, C, Bn), lambda b, n: (b, 0, n)),
            pl.BlockSpec((1, 4, Bn), lambda b, n: (b, 0, n)),
        ],
        out_specs=[
            pl.BlockSpec((1, CP, Bn), lambda b, n: (b, 0, n)),
            pl.BlockSpec((1, NPLANES, Bn), lambda b, n: (b, 0, n)),
        ],
        out_shape=[
            jax.ShapeDtypeStruct((nbatch, CP, npts), jnp.float32),
            jax.ShapeDtypeStruct((nbatch, NPLANES, npts), jnp.int32),
        ],
    )(consts, gr, p4)

    TBn = 512
    wpad4 = wpad.reshape(nbatch, 2, HALF, npts)
    wt = pl.pallas_call(
        _tr_body,
        grid=(2, nbatch, npts // TBn),
        in_specs=[
            pl.BlockSpec((1, 1, HALF, TBn), lambda h, b, n: (b, h, 0, n)),
        ],
        out_specs=pl.BlockSpec((1, 1, TBn, HALF), lambda h, b, n: (h, b, n, 0)),
        out_shape=jax.ShapeDtypeStruct((2, nbatch, npts, HALF), jnp.float32),
    )(wpad4)
    idx4 = idx.reshape(nbatch, NPLANES, 1, npts)
    zeros = jnp.zeros((CELLS, HALF), jnp.float32)

    accs = _make_sc_scatter(nbatch, npts)(wt, idx4, zeros)

    Bc = 1024
    nr = nbatch * NPLANES
    a0 = accs[0].reshape(nr, CELLS, HALF)
    a1 = accs[1].reshape(nr, CELLS, HALF)
    o0, o1 = pl.pallas_call(
        _post_body,
        grid=(nr, CELLS // Bc),
        in_specs=[
            pl.BlockSpec((1, Bc, HALF), lambda r, c: (r, c, 0)),
            pl.BlockSpec((1, Bc, HALF), lambda r, c: (r, c, 0)),
        ],
        out_specs=[
            pl.BlockSpec((1, Bc, HALF), lambda r, c: (r, c, 0)),
            pl.BlockSpec((1, Bc, C - HALF), lambda r, c: (r, c, 0)),
        ],
        out_shape=[
            jax.ShapeDtypeStruct((nr, CELLS, HALF), jnp.float32),
            jax.ShapeDtypeStruct((nr, CELLS, C - HALF), jnp.float32),
        ],
    )(a0, a1)

    out = jnp.concatenate([o0, o1], axis=-1)
    out = out.reshape(nbatch, NPLANES, CELLS, C)
    out = jnp.transpose(out, (0, 1, 3, 2))
    return out.reshape(nbatch, NPLANES, C, GRID, GRID)


# speedup vs baseline: 1.1041x; 1.1041x over previous
import functools

import jax
import jax.numpy as jnp
from jax import lax
from jax.experimental import pallas as pl
from jax.experimental.pallas import tpu as pltpu
from jax.experimental.pallas import tpu_sc as plsc

GRID = 128
CELLS = GRID * GRID
C = 196
CP = 208
HALF = CP // 2
NPLANES = 3
NSUB = 16
PCHUNK = 64


def _pre_body(consts_ref, gr_ref, p_ref, w_ref, idx_ref):
    Bn = gr_ref.shape[2]
    npts = pl.num_programs(1) * Bn
    rmod = npts % C
    n0 = pl.program_id(1) * Bn
    a = gr_ref[0]
    ci = lax.broadcasted_iota(jnp.int32, (C, Bn), 0)
    ni = n0 + lax.broadcasted_iota(jnp.int32, (C, Bn), 1)
    q = lax.rem(ni + rmod * ci, C)
    s0 = consts_ref[0]
    o0 = consts_ref[1]
    s1 = consts_ref[2]
    o1 = consts_ref[3]
    s2 = consts_ref[4]
    o2 = consts_ref[5]
    bv = jnp.where(q == 0, a * s0 + o0,
         jnp.where(q == 1, a * s1 + o1,
         jnp.where(q == 2, a * s2 + o2, a)))
    p4 = p_ref[0]
    alpha = jax.nn.sigmoid(p4[3:4, :])
    w_ref[0, 0:C, :] = bv * alpha
    w_ref[0, C:C + 1, :] = alpha
    w_ref[0, C + 1:CP, :] = jnp.zeros((CP - C - 1, Bn), jnp.float32)

    x = p4[0:1, :] * s0 + o0
    y = p4[1:2, :] * s1 + o1
    z = p4[2:3, :] * s2 + o2

    def cell(u):
        g = ((u * 0.5 + 0.5) * (GRID - 1)).astype(jnp.int32)
        return jnp.clip(g, 0, GRID - 1)

    gx, gy, gz = cell(x), cell(y), cell(z)
    idx_ref[0, 0:1, :] = gx * GRID + gy
    idx_ref[0, 1:2, :] = gx * GRID + gz
    idx_ref[0, 2:3, :] = gy * GRID + gz


def _tr_body(w_ref, t_ref):
    a = w_ref[0, 0]
    apad = jnp.concatenate(
        [a, jnp.zeros((128 - HALF, a.shape[1]), jnp.float32)], axis=0)
    t = jnp.transpose(apad)
    t_ref[0, 0] = t[:, 0:HALF]


def _post_body(a0_ref, a1_ref, o0_ref, o1_ref):
    m0 = a0_ref[0]
    m1 = a1_ref[0]
    w = jnp.maximum(m1[:, C - HALF:C - HALF + 1], 1e-6)
    o0_ref[0] = m0 / w
    o1_ref[0] = m1[:, 0:C - HALF] / w


def _make_sc_scatter(nbatch, npts):
    pts_per_sub = npts // NSUB
    nchunks = pts_per_sub // PCHUNK
    rows_per_sub = CELLS // NSUB
    nrounds = nbatch * NPLANES
    mesh = plsc.VectorSubcoreMesh(core_axis_name="c", subcore_axis_name="s")

    npairs = nchunks // 2

    @functools.partial(
        pl.kernel,
        mesh=mesh,
        compiler_params=pltpu.CompilerParams(use_tc_tiling_on_sc=False),
        out_type=jax.ShapeDtypeStruct((2, nbatch, NPLANES, CELLS, HALF),
                                      jnp.float32),
        scratch_types=[
            pltpu.VMEM_SHARED((CELLS, HALF), jnp.float32),
            pltpu.VMEM((PCHUNK,), jnp.int32),
            pltpu.VMEM((PCHUNK, HALF), jnp.float32),
            pltpu.VMEM((PCHUNK,), jnp.int32),
            pltpu.VMEM((PCHUNK, HALF), jnp.float32),
            pltpu.SemaphoreType.DMA,
            pltpu.SemaphoreType.DMA,
        ],
    )
    def sc_scatter(wt_hbm, idx_hbm, z_hbm, out_hbm, acc,
                   idxa, rowsa, idxb, rowsb, sema, semb):
        cid = lax.axis_index("c")
        sid = lax.axis_index("s")
        r0 = sid * rows_per_sub
        pbase = sid * pts_per_sub

        def load(k, idxv, rows, sem, b, p):
            base = pbase + k * PCHUNK
            pltpu.async_copy(idx_hbm.at[b, p, 0, pl.ds(base, PCHUNK)],
                             idxv, sem)
            pltpu.async_copy(wt_hbm.at[cid, b, pl.ds(base, PCHUNK), :],
                             rows, sem)

        def drain(idxv, rows, sem, b, p):
            pltpu.make_async_copy(idx_hbm.at[b, p, 0, pl.ds(pbase, PCHUNK)],
                                  idxv, sem).wait()
            pltpu.make_async_copy(wt_hbm.at[cid, b, pl.ds(pbase, PCHUNK), :],
                                  rows, sem).wait()

        def round_body(r, carry):
            b = r // NPLANES
            p = lax.rem(r, NPLANES)
            pltpu.sync_copy(z_hbm.at[pl.ds(r0, rows_per_sub), :],
                            acc.at[pl.ds(r0, rows_per_sub), :])
            plsc.subcore_barrier()
            load(0, idxa, rowsa, sema, b, p)

            def pair(i, carry2):
                load(2 * i + 1, idxb, rowsb, semb, b, p)
                drain(idxa, rowsa, sema, b, p)
                pltpu.sync_copy(rowsa, acc.at[idxa], add=True)

                @pl.when(i + 1 < npairs)
                def _():
                    load(2 * i + 2, idxa, rowsa, sema, b, p)

                drain(idxb, rowsb, semb, b, p)
                pltpu.sync_copy(rowsb, acc.at[idxb], add=True)
                return carry2

            lax.fori_loop(0, npairs, pair, 0)
            plsc.subcore_barrier()
            pltpu.sync_copy(
                acc.at[pl.ds(r0, rows_per_sub), :],
                out_hbm.at[cid, b, p, pl.ds(r0, rows_per_sub), :])
            return carry

        lax.fori_loop(0, nrounds, round_body, 0)

    return sc_scatter


def kernel(GS_feats, scene_bounds):
    nbatch, npts, nchan = GS_feats.shape
    sb = scene_bounds.astype(jnp.float32)
    s0 = 2.0 / (sb[1] - sb[0])
    o0 = -2.0 * sb[0] / (sb[1] - sb[0]) - 1.0
    s1 = 2.0 / (sb[3] - sb[2])
    o1 = -2.0 * sb[2] / (sb[3] - sb[2]) - 1.0
    s2 = 2.0 / (sb[5] - sb[4])
    o2 = -2.0 * sb[4] / (sb[5] - sb[4]) - 1.0
    consts = jnp.stack([s0, o0, s1, o1, s2, o2,
                        jnp.float32(0.0), jnp.float32(0.0)])

    gr = GS_feats.reshape(nbatch, nchan, npts)
    p4 = jnp.transpose(GS_feats[:, :, 0:4], (0, 2, 1))

    Bn = 512
    wpad, idx = pl.pallas_call(
        _pre_body,
        grid=(nbatch, npts // Bn),
        in_specs=[
            pl.BlockSpec(memory_space=pltpu.SMEM),
            pl.BlockSpec((1, C, Bn), lambda b, n: (b, 0, n)),
            pl.BlockSpec((1, 4, Bn), lambda b, n: (b, 0, n)),
        ],
        out_specs=[
            pl.BlockSpec((1, CP, Bn), lambda b, n: (b, 0, n)),
            pl.BlockSpec((1, NPLANES, Bn), lambda b, n: (b, 0, n)),
        ],
        out_shape=[
            jax.ShapeDtypeStruct((nbatch, CP, npts), jnp.float32),
            jax.ShapeDtypeStruct((nbatch, NPLANES, npts), jnp.int32),
        ],
    )(consts, gr, p4)

    wt = jnp.transpose(wpad.reshape(nbatch, 2, HALF, npts), (1, 0, 3, 2))
    idx4 = idx.reshape(nbatch, NPLANES, 1, npts)
    zeros = jnp.zeros((CELLS, HALF), jnp.float32)

    accs = _make_sc_scatter(nbatch, npts)(wt, idx4, zeros)

    Bc = 1024
    nr = nbatch * NPLANES
    a0 = accs[0].reshape(nr, CELLS, HALF)
    a1 = accs[1].reshape(nr, CELLS, HALF)
    o0, o1 = pl.pallas_call(
        _post_body,
        grid=(nr, CELLS // Bc),
        in_specs=[
            pl.BlockSpec((1, Bc, HALF), lambda r, c: (r, c, 0)),
            pl.BlockSpec((1, Bc, HALF), lambda r, c: (r, c, 0)),
        ],
        out_specs=[
            pl.BlockSpec((1, Bc, HALF), lambda r, c: (r, c, 0)),
            pl.BlockSpec((1, Bc, C - HALF), lambda r, c: (r, c, 0)),
        ],
        out_shape=[
            jax.ShapeDtypeStruct((nr, CELLS, HALF), jnp.float32),
            jax.ShapeDtypeStruct((nr, CELLS, C - HALF), jnp.float32),
        ],
    )(a0, a1)

    out = jnp.concatenate([o0, o1], axis=-1)
    out = out.reshape(nbatch, NPLANES, CELLS, C)
    out = jnp.transpose(out, (0, 1, 3, 2))
    return out.reshape(nbatch, NPLANES, C, GRID, GRID)


# speedup vs baseline: 1.5002x; 1.3587x over previous
import functools

import jax
import jax.numpy as jnp
from jax import lax
from jax.experimental import pallas as pl
from jax.experimental.pallas import tpu as pltpu
from jax.experimental.pallas import tpu_sc as plsc

GRID = 128
CELLS = GRID * GRID
C = 196
CP = 208
HALF = CP // 2
NPLANES = 3
NSUB = 16
PCHUNK = 64
LN = 1024


def _idx_body(consts_ref, x_ref, y_ref, z_ref, i0_ref, i1_ref, i2_ref):
    s0 = consts_ref[0]
    o0 = consts_ref[1]
    s1 = consts_ref[2]
    o1 = consts_ref[3]
    s2 = consts_ref[4]
    o2 = consts_ref[5]

    def cell(u):
        g = ((u * 0.5 + 0.5) * (GRID - 1)).astype(jnp.int32)
        return jnp.clip(g, 0, GRID - 1)

    gx = cell(x_ref[0, 0] * s0 + o0)
    gy = cell(y_ref[0, 0] * s1 + o1)
    gz = cell(z_ref[0, 0] * s2 + o2)
    i0_ref[0] = gx * GRID + gy
    i1_ref[0] = gx * GRID + gz
    i2_ref[0] = gy * GRID + gz


def _w_body(rmod, consts_ref, a_ref, *refs):
    f_refs = refs[:8]
    w_ref = refs[8]
    cg = pl.program_id(1)
    alpha = jax.nn.sigmoid(a_ref[0, 0])
    nrow, ln = alpha.shape
    n2 = (lax.broadcasted_iota(jnp.int32, (nrow, ln), 0) * ln
          + lax.broadcasted_iota(jnp.int32, (nrow, ln), 1))
    s0 = consts_ref[0]
    o0 = consts_ref[1]
    s1 = consts_ref[2]
    o1 = consts_ref[3]
    s2 = consts_ref[4]
    o2 = consts_ref[5]
    for i in range(8):
        c = cg * 8 + i
        v = f_refs[i][0, 0]
        q = lax.rem(n2 + rmod * c, C)
        v = jnp.where(q == 0, v * s0 + o0,
            jnp.where(q == 1, v * s1 + o1,
            jnp.where(q == 2, v * s2 + o2, v)))
        v = jnp.where(c < C, v * alpha,
            jnp.where(c == C, alpha, jnp.zeros_like(v)))
        w_ref[0, i] = v


def _post_body(a0_ref, a1_ref, o0_ref, o1_ref):
    m0 = a0_ref[0]
    m1 = a1_ref[0]
    w = jnp.maximum(m1[:, C - HALF:C - HALF + 1], 1e-6)
    o0_ref[0] = m0 / w
    o1_ref[0] = m1[:, 0:C - HALF] / w


def _make_sc_scatter(nbatch, npts):
    pts_per_sub = npts // NSUB
    nchunks = pts_per_sub // PCHUNK
    rows_per_sub = CELLS // NSUB
    npairs = nchunks // 2
    mesh = plsc.VectorSubcoreMesh(core_axis_name="c", subcore_axis_name="s")

    @functools.partial(
        pl.kernel,
        mesh=mesh,
        compiler_params=pltpu.CompilerParams(use_tc_tiling_on_sc=False),
        out_type=jax.ShapeDtypeStruct((2, nbatch, NPLANES, CELLS, HALF),
                                      jnp.float32),
        scratch_types=[
            pltpu.VMEM_SHARED((CELLS, HALF), jnp.float32),
            pltpu.VMEM((PCHUNK,), jnp.int32),
            pltpu.VMEM((PCHUNK, HALF), jnp.float32),
            pltpu.VMEM((PCHUNK,), jnp.int32),
            pltpu.VMEM((PCHUNK, HALF), jnp.float32),
            pltpu.SemaphoreType.DMA,
            pltpu.SemaphoreType.DMA,
        ],
    )
    def sc_scatter(wt_hbm, i0_hbm, i1_hbm, i2_hbm, z_hbm, out_hbm, acc,
                   idxa, rowsa, idxb, rowsb, sema, semb):
        cid = lax.axis_index("c")
        sid = lax.axis_index("s")
        r0 = sid * rows_per_sub
        pbase = sid * pts_per_sub

        def plane_round(p, idx_hbm):
            def load(k, idxv, rows, sem, b):
                base = pbase + k * PCHUNK
                pltpu.async_copy(idx_hbm.at[b, pl.ds(base, PCHUNK)],
                                 idxv, sem)
                pltpu.async_copy(wt_hbm.at[cid, b, pl.ds(base, PCHUNK), :],
                                 rows, sem)

            def drain(idxv, rows, sem, b):
                pltpu.make_async_copy(idx_hbm.at[b, pl.ds(pbase, PCHUNK)],
                                      idxv, sem).wait()
                pltpu.make_async_copy(
                    wt_hbm.at[cid, b, pl.ds(pbase, PCHUNK), :],
                    rows, sem).wait()

            def round_body(b, carry):
                pltpu.sync_copy(z_hbm.at[pl.ds(r0, rows_per_sub), :],
                                acc.at[pl.ds(r0, rows_per_sub), :])
                plsc.subcore_barrier()
                load(0, idxa, rowsa, sema, b)

                def pair(i, carry2):
                    load(2 * i + 1, idxb, rowsb, semb, b)
                    drain(idxa, rowsa, sema, b)
                    pltpu.sync_copy(rowsa, acc.at[idxa], add=True)

                    @pl.when(i + 1 < npairs)
                    def _():
                        load(2 * i + 2, idxa, rowsa, sema, b)

                    drain(idxb, rowsb, semb, b)
                    pltpu.sync_copy(rowsb, acc.at[idxb], add=True)
                    return carry2

                lax.fori_loop(0, npairs, pair, 0)
                plsc.subcore_barrier()
                pltpu.sync_copy(
                    acc.at[pl.ds(r0, rows_per_sub), :],
                    out_hbm.at[cid, b, p, pl.ds(r0, rows_per_sub), :])
                return carry

            lax.fori_loop(0, nbatch, round_body, 0)

        plane_round(0, i0_hbm)
        plane_round(1, i1_hbm)
        plane_round(2, i2_hbm)

    return sc_scatter


def kernel(GS_feats, scene_bounds):
    nbatch, npts, nchan = GS_feats.shape
    nrow = npts // LN
    sb = scene_bounds.astype(jnp.float32)
    s0 = 2.0 / (sb[1] - sb[0])
    o0 = -2.0 * sb[0] / (sb[1] - sb[0]) - 1.0
    s1 = 2.0 / (sb[3] - sb[2])
    o1 = -2.0 * sb[2] / (sb[3] - sb[2]) - 1.0
    s2 = 2.0 / (sb[5] - sb[4])
    o2 = -2.0 * sb[4] / (sb[5] - sb[4]) - 1.0
    consts = jnp.stack([s0, o0, s1, o1, s2, o2,
                        jnp.float32(0.0), jnp.float32(0.0)])

    p43 = jnp.transpose(GS_feats[:, :, 0:4], (0, 2, 1)).reshape(
        nbatch, 4, nrow, LN)
    flat3 = GS_feats.reshape(nbatch, (npts * nchan) // LN, LN)

    NB = 8
    nbr = nrow // NB
    i0, i1, i2 = pl.pallas_call(
        _idx_body,
        grid=(nbatch, nbr),
        in_specs=[
            pl.BlockSpec(memory_space=pltpu.SMEM),
            pl.BlockSpec((1, 1, NB, LN), lambda b, n: (b, 0, n, 0)),
            pl.BlockSpec((1, 1, NB, LN), lambda b, n: (b, 1, n, 0)),
            pl.BlockSpec((1, 1, NB, LN), lambda b, n: (b, 2, n, 0)),
        ],
        out_specs=[
            pl.BlockSpec((1, NB, LN), lambda b, n: (b, n, 0)),
            pl.BlockSpec((1, NB, LN), lambda b, n: (b, n, 0)),
            pl.BlockSpec((1, NB, LN), lambda b, n: (b, n, 0)),
        ],
        out_shape=[
            jax.ShapeDtypeStruct((nbatch, nrow, LN), jnp.int32),
            jax.ShapeDtypeStruct((nbatch, nrow, LN), jnp.int32),
            jax.ShapeDtypeStruct((nbatch, nrow, LN), jnp.int32),
        ],
    )(consts, p43, p43, p43)

    ngroups = CP // 8
    flat4 = flat3.reshape(nbatch, nchan, nrow, LN)
    wpad = pl.pallas_call(
        functools.partial(_w_body, npts % nchan),
        grid=(nbatch, ngroups),
        in_specs=[
            pl.BlockSpec(memory_space=pltpu.SMEM),
            pl.BlockSpec((1, 1, nrow, LN), lambda b, g: (b, 3, 0, 0)),
        ] + [
            pl.BlockSpec((1, 1, nrow, LN),
                         functools.partial(
                             lambda b, g, i=0:
                             (b, jnp.minimum(g * 8 + i, C - 1), 0, 0),
                             i=i))
            for i in range(8)
        ],
        out_specs=pl.BlockSpec((1, 8, nrow, LN), lambda b, g: (b, g, 0, 0)),
        out_shape=jax.ShapeDtypeStruct((nbatch, CP, nrow, LN), jnp.float32),
    )(consts, p43, *([flat4] * 8))

    wt = jnp.transpose(
        wpad.reshape(nbatch, 2, HALF, npts), (1, 0, 3, 2))
    zeros = jnp.zeros((CELLS, HALF), jnp.float32)
    i0f = i0.reshape(nbatch, npts)
    i1f = i1.reshape(nbatch, npts)
    i2f = i2.reshape(nbatch, npts)

    accs = _make_sc_scatter(nbatch, npts)(wt, i0f, i1f, i2f, zeros)

    Bc = 1024
    nr = nbatch * NPLANES
    a0 = accs[0].reshape(nr, CELLS, HALF)
    a1 = accs[1].reshape(nr, CELLS, HALF)
    o0_, o1_ = pl.pallas_call(
        _post_body,
        grid=(nr, CELLS // Bc),
        in_specs=[
            pl.BlockSpec((1, Bc, HALF), lambda r, c2: (r, c2, 0)),
            pl.BlockSpec((1, Bc, HALF), lambda r, c2: (r, c2, 0)),
        ],
        out_specs=[
            pl.BlockSpec((1, Bc, HALF), lambda r, c2: (r, c2, 0)),
            pl.BlockSpec((1, Bc, C - HALF), lambda r, c2: (r, c2, 0)),
        ],
        out_shape=[
            jax.ShapeDtypeStruct((nr, CELLS, HALF), jnp.float32),
            jax.ShapeDtypeStruct((nr, CELLS, C - HALF), jnp.float32),
        ],
    )(a0, a1)

    out = jnp.concatenate([o0_, o1_], axis=-1)
    out = out.reshape(nbatch, NPLANES, CELLS, C)
    out = jnp.transpose(out, (0, 1, 3, 2))
    return out.reshape(nbatch, NPLANES, C, GRID, GRID)


# speedup vs baseline: 1.7218x; 1.1477x over previous
import functools

import jax
import jax.numpy as jnp
from jax import lax
from jax.experimental import pallas as pl
from jax.experimental.pallas import tpu as pltpu
from jax.experimental.pallas import tpu_sc as plsc

GRID = 128
CELLS = GRID * GRID
C = 196
CP = 208
HALF = CP // 2
NPLANES = 3
NSUB = 16
PCHUNK = 64
LN = 1024


def _idx_body(consts_ref, x_ref, y_ref, z_ref, i0_ref, i1_ref, i2_ref):
    s0 = consts_ref[0]
    o0 = consts_ref[1]
    s1 = consts_ref[2]
    o1 = consts_ref[3]
    s2 = consts_ref[4]
    o2 = consts_ref[5]

    def cell(u):
        g = ((u * 0.5 + 0.5) * (GRID - 1)).astype(jnp.int32)
        return jnp.clip(g, 0, GRID - 1)

    gx = cell(x_ref[0, 0] * s0 + o0)
    gy = cell(y_ref[0, 0] * s1 + o1)
    gz = cell(z_ref[0, 0] * s2 + o2)
    i0_ref[0] = gx * GRID + gy
    i1_ref[0] = gx * GRID + gz
    i2_ref[0] = gy * GRID + gz


def _w_body(rmod, consts_ref, a_ref, *refs):
    f_refs = refs[:8]
    w_ref = refs[8]
    cg = pl.program_id(1)
    alpha = jax.nn.sigmoid(a_ref[0, 0])
    nrow, ln = alpha.shape
    n2 = (lax.broadcasted_iota(jnp.int32, (nrow, ln), 0) * ln
          + lax.broadcasted_iota(jnp.int32, (nrow, ln), 1))
    s0 = consts_ref[0]
    o0 = consts_ref[1]
    s1 = consts_ref[2]
    o1 = consts_ref[3]
    s2 = consts_ref[4]
    o2 = consts_ref[5]
    for i in range(8):
        c = cg * 8 + i
        v = f_refs[i][0, 0]
        q = lax.rem(n2 + rmod * c, C)
        v = jnp.where(q == 0, v * s0 + o0,
            jnp.where(q == 1, v * s1 + o1,
            jnp.where(q == 2, v * s2 + o2, v)))
        v = jnp.where(c < C, v * alpha,
            jnp.where(c == C, alpha, jnp.zeros_like(v)))
        w_ref[0, i] = v


def _post_body(a0_ref, a1_ref, out_ref):
    m0 = a0_ref[0]
    m1 = a1_ref[0]
    w = jnp.maximum(m1[:, C - HALF:C - HALF + 1], 1e-6)
    t0 = jnp.transpose(m0 / w)
    t1 = jnp.transpose(m1[:, 0:96] / w)[0:C - HALF, :]
    out_ref[0, 0] = jnp.concatenate([t0, t1], axis=0)


def _make_sc_scatter(nbatch, npts):
    pts_per_sub = npts // NSUB
    nchunks = pts_per_sub // PCHUNK
    rows_per_sub = CELLS // NSUB
    npairs = nchunks // 2
    mesh = plsc.VectorSubcoreMesh(core_axis_name="c", subcore_axis_name="s")

    @functools.partial(
        pl.kernel,
        mesh=mesh,
        compiler_params=pltpu.CompilerParams(use_tc_tiling_on_sc=False),
        out_type=jax.ShapeDtypeStruct((2, nbatch, NPLANES, CELLS, HALF),
                                      jnp.float32),
        scratch_types=[
            pltpu.VMEM_SHARED((CELLS, HALF), jnp.float32),
            pltpu.VMEM((PCHUNK,), jnp.int32),
            pltpu.VMEM((PCHUNK, HALF), jnp.float32),
            pltpu.VMEM((PCHUNK,), jnp.int32),
            pltpu.VMEM((PCHUNK, HALF), jnp.float32),
            pltpu.SemaphoreType.DMA,
            pltpu.SemaphoreType.DMA,
        ],
    )
    def sc_scatter(wt_hbm, i0_hbm, i1_hbm, i2_hbm, z_hbm, out_hbm, acc,
                   idxa, rowsa, idxb, rowsb, sema, semb):
        cid = lax.axis_index("c")
        sid = lax.axis_index("s")
        r0 = sid * rows_per_sub
        pbase = sid * pts_per_sub

        def plane_round(p, idx_hbm):
            def load(k, idxv, rows, sem, b):
                base = pbase + k * PCHUNK
                pltpu.async_copy(idx_hbm.at[b, pl.ds(base, PCHUNK)],
                                 idxv, sem)
                pltpu.async_copy(wt_hbm.at[cid, b, pl.ds(base, PCHUNK), :],
                                 rows, sem)

            def drain(idxv, rows, sem, b):
                pltpu.make_async_copy(idx_hbm.at[b, pl.ds(pbase, PCHUNK)],
                                      idxv, sem).wait()
                pltpu.make_async_copy(
                    wt_hbm.at[cid, b, pl.ds(pbase, PCHUNK), :],
                    rows, sem).wait()

            def round_body(b, carry):
                pltpu.sync_copy(z_hbm.at[pl.ds(r0, rows_per_sub), :],
                                acc.at[pl.ds(r0, rows_per_sub), :])
                plsc.subcore_barrier()
                load(0, idxa, rowsa, sema, b)

                def pair(i, carry2):
                    load(2 * i + 1, idxb, rowsb, semb, b)
                    drain(idxa, rowsa, sema, b)
                    pltpu.sync_copy(rowsa, acc.at[idxa], add=True)

                    @pl.when(i + 1 < npairs)
                    def _():
                        load(2 * i + 2, idxa, rowsa, sema, b)

                    drain(idxb, rowsb, semb, b)
                    pltpu.sync_copy(rowsb, acc.at[idxb], add=True)
                    return carry2

                lax.fori_loop(0, npairs, pair, 0)
                plsc.subcore_barrier()
                pltpu.sync_copy(
                    acc.at[pl.ds(r0, rows_per_sub), :],
                    out_hbm.at[cid, b, p, pl.ds(r0, rows_per_sub), :])
                return carry

            lax.fori_loop(0, nbatch, round_body, 0)

        plane_round(0, i0_hbm)
        plane_round(1, i1_hbm)
        plane_round(2, i2_hbm)

    return sc_scatter


def kernel(GS_feats, scene_bounds):
    nbatch, npts, nchan = GS_feats.shape
    nrow = npts // LN
    sb = scene_bounds.astype(jnp.float32)
    s0 = 2.0 / (sb[1] - sb[0])
    o0 = -2.0 * sb[0] / (sb[1] - sb[0]) - 1.0
    s1 = 2.0 / (sb[3] - sb[2])
    o1 = -2.0 * sb[2] / (sb[3] - sb[2]) - 1.0
    s2 = 2.0 / (sb[5] - sb[4])
    o2 = -2.0 * sb[4] / (sb[5] - sb[4]) - 1.0
    consts = jnp.stack([s0, o0, s1, o1, s2, o2,
                        jnp.float32(0.0), jnp.float32(0.0)])

    p43 = jnp.transpose(GS_feats[:, :, 0:4], (0, 2, 1)).reshape(
        nbatch, 4, nrow, LN)
    flat3 = GS_feats.reshape(nbatch, (npts * nchan) // LN, LN)

    NB = 8
    nbr = nrow // NB
    i0, i1, i2 = pl.pallas_call(
        _idx_body,
        grid=(nbatch, nbr),
        in_specs=[
            pl.BlockSpec(memory_space=pltpu.SMEM),
            pl.BlockSpec((1, 1, NB, LN), lambda b, n: (b, 0, n, 0)),
            pl.BlockSpec((1, 1, NB, LN), lambda b, n: (b, 1, n, 0)),
            pl.BlockSpec((1, 1, NB, LN), lambda b, n: (b, 2, n, 0)),
        ],
        out_specs=[
            pl.BlockSpec((1, NB, LN), lambda b, n: (b, n, 0)),
            pl.BlockSpec((1, NB, LN), lambda b, n: (b, n, 0)),
            pl.BlockSpec((1, NB, LN), lambda b, n: (b, n, 0)),
        ],
        out_shape=[
            jax.ShapeDtypeStruct((nbatch, nrow, LN), jnp.int32),
            jax.ShapeDtypeStruct((nbatch, nrow, LN), jnp.int32),
            jax.ShapeDtypeStruct((nbatch, nrow, LN), jnp.int32),
        ],
    )(consts, p43, p43, p43)

    ngroups = CP // 8
    flat4 = flat3.reshape(nbatch, nchan, nrow, LN)
    wpad = pl.pallas_call(
        functools.partial(_w_body, npts % nchan),
        grid=(nbatch, ngroups),
        in_specs=[
            pl.BlockSpec(memory_space=pltpu.SMEM),
            pl.BlockSpec((1, 1, nrow, LN), lambda b, g: (b, 3, 0, 0)),
        ] + [
            pl.BlockSpec((1, 1, nrow, LN),
                         functools.partial(
                             lambda b, g, i=0:
                             (b, jnp.minimum(g * 8 + i, C - 1), 0, 0),
                             i=i))
            for i in range(8)
        ],
        out_specs=pl.BlockSpec((1, 8, nrow, LN), lambda b, g: (b, g, 0, 0)),
        out_shape=jax.ShapeDtypeStruct((nbatch, CP, nrow, LN), jnp.float32),
    )(consts, p43, *([flat4] * 8))

    wt = jnp.transpose(
        wpad.reshape(nbatch, 2, HALF, nrow, LN),
        (1, 0, 3, 4, 2)).reshape(2, nbatch, npts, HALF)
    zeros = jnp.zeros((CELLS, HALF), jnp.float32)
    i0f = i0.reshape(nbatch, npts)
    i1f = i1.reshape(nbatch, npts)
    i2f = i2.reshape(nbatch, npts)

    accs = _make_sc_scatter(nbatch, npts)(wt, i0f, i1f, i2f, zeros)

    Bc = 1024
    nr = nbatch * NPLANES
    a0 = accs[0].reshape(nr, CELLS, HALF)
    a1 = accs[1].reshape(nr, CELLS, HALF)
    out = pl.pallas_call(
        _post_body,
        grid=(nr, CELLS // Bc),
        in_specs=[
            pl.BlockSpec((1, Bc, HALF), lambda r, c2: (r, c2, 0)),
            pl.BlockSpec((1, Bc, HALF), lambda r, c2: (r, c2, 0)),
        ],
        out_specs=pl.BlockSpec((1, 1, C, Bc), lambda r, c2: (r, 0, 0, c2)),
        out_shape=jax.ShapeDtypeStruct((nr, 1, C, CELLS), jnp.float32),
    )(a0, a1)

    return out.reshape(nbatch, NPLANES, C, GRID, GRID)
